# baseline (speedup 1.0000x reference)
import functools

import jax
import jax.numpy as jnp
from jax.experimental import pallas as pl

GROUPS = 16
EPS = 1e-06
BETA = 0.25

_INTERPRET = False


def _mm(a, b):
    return jax.lax.dot_general(
        a.astype(jnp.bfloat16), b.astype(jnp.bfloat16),
        (((1,), (0,)), ((), ())), preferred_element_type=jnp.float32)


def _ceil_to(a, b):
    return -(-a // b) * b


def _stats_body(x_ref, o_ref):
    i = pl.program_id(0)

    @pl.when(i == 0)
    def _():
        o_ref[...] = jnp.zeros_like(o_ref)

    x = x_ref[...]
    o_ref[0:1, :] = o_ref[0:1, :] + jnp.sum(x, axis=0, keepdims=True)
    o_ref[1:2, :] = o_ref[1:2, :] + jnp.sum(x * x, axis=0, keepdims=True)


def _gn_body(x_ref, st_ref, g_ref, gb_ref, o_ref, *, gs, n_int, cin, silu):
    s1 = st_ref[0:1, :]
    s2 = st_ref[1:2, :]
    r = jax.lax.broadcasted_iota(jnp.int32, (cin, cin), 0) // gs
    c = jax.lax.broadcasted_iota(jnp.int32, (cin, cin), 1) // gs
    avg = jnp.where(r == c, 1.0 / (gs * n_int), 0.0).astype(jnp.float32)
    m = jax.lax.dot_general(s1, avg, (((1,), (0,)), ((), ())),
                            precision=jax.lax.Precision.HIGHEST,
                            preferred_element_type=jnp.float32)
    ex2 = jax.lax.dot_general(s2, avg, (((1,), (0,)), ((), ())),
                              precision=jax.lax.Precision.HIGHEST,
                              preferred_element_type=jnp.float32)
    var = ex2 - m * m
    y = (x_ref[...] - m) * jax.lax.rsqrt(var + EPS) * g_ref[...] + gb_ref[...]
    if silu:
        y = y * jax.nn.sigmoid(y)
    o_ref[...] = y


def _gn_apply(x2d, gn, silu):
    n_int, cin = x2d.shape
    g, gb = gn
    T = min(4096, _ceil_to(n_int, 8))
    ntiles = -(-n_int // T)
    npad = ntiles * T
    xp = jnp.pad(x2d, ((0, npad - n_int), (0, 0)))
    stats = pl.pallas_call(
        _stats_body,
        grid=(ntiles,),
        in_specs=[pl.BlockSpec((T, cin), lambda i: (i, 0))],
        out_specs=pl.BlockSpec((8, cin), lambda i: (0, 0)),
        out_shape=jax.ShapeDtypeStruct((8, cin), jnp.float32),
        interpret=_INTERPRET,
    )(xp)
    y = pl.pallas_call(
        functools.partial(_gn_body, gs=cin // GROUPS, n_int=n_int,
                          cin=cin, silu=silu),
        grid=(ntiles,),
        in_specs=[pl.BlockSpec((T, cin), lambda i: (i, 0)),
                  pl.BlockSpec((8, cin), lambda i: (0, 0)),
                  pl.BlockSpec((1, cin), lambda i: (0, 0)),
                  pl.BlockSpec((1, cin), lambda i: (0, 0))],
        out_specs=pl.BlockSpec((T, cin), lambda i: (i, 0)),
        out_shape=jax.ShapeDtypeStruct((npad, cin), jnp.float32),
        interpret=_INTERPRET,
    )(xp, stats, g.reshape(1, cin), gb.reshape(1, cin))
    return y[:n_int]


def _conv_body(x_ref, w_ref, b_ref, *rest, cfg):
    T, Tw, center, cin, cout = (cfg['T'], cfg['Tw'], cfg['center'],
                                cfg['cin'], cfg['cout'])
    rest = list(rest)
    out_ref = rest.pop()
    i = pl.program_id(0)
    xw = x_ref[pl.ds(i * T, Tw), :]

    Hp, Wp = cfg['Hp'], cfg['Wp']
    rels = [kd * Hp * Wp + kh * Wp + kw
            for kd in (-1, 0, 1) for kh in (-1, 0, 1) for kw in (-1, 0, 1)]
    acc = None
    for t, rel in enumerate(rels):
        xs = jax.lax.slice(xw, (center + rel, 0), (center + rel + T, cin))
        if cin == 1:
            xb = xs.astype(jnp.bfloat16).astype(jnp.float32)
            wb = w_ref[t].astype(jnp.bfloat16).astype(jnp.float32)
            part = xb * wb
        else:
            part = _mm(xs, w_ref[t])
        acc = part if acc is None else acc + part
    acc = acc + b_ref[...]

    if cfg['res_mode'] == 1:
        acc = acc + rest[0][...]
    elif cfg['res_mode'] == 2:
        acc = acc + _mm(rest[0][...], rest[1][...]) + rest[2][...]

    out_ref[...] = acc


def _run_conv(x, wb, gn=None, silu=False, res=None, res_wb=None, mode='same'):
    D, H, W, cin = x.shape
    Dp, Hp, Wp = D + 2, H + 2, W + 2
    Np = Dp * Hp * Wp
    center = Hp * Wp + Wp + 1
    w, b = wb
    cout = w.shape[0]

    if gn is not None:
        x = _gn_apply(x.reshape(D * H * W, cin), gn, silu).reshape(x.shape)

    T = min(4096, _ceil_to(Np, 8))
    ntiles = -(-Np // T)
    np_pad = ntiles * T
    Tw = _ceil_to(T + 2 * center, 8)
    L = (ntiles - 1) * T + Tw

    xf = jnp.pad(x, ((1, 1), (1, 1), (1, 1), (0, 0))).reshape(Np, cin)
    xe = jnp.pad(xf, ((center, L - center - Np), (0, 0)))
    wt = jnp.transpose(w, (2, 3, 4, 1, 0)).reshape(27, cin, cout)

    args = [xe, wt, b.reshape(1, cout)]
    in_specs = [pl.BlockSpec((L, cin), lambda i: (0, 0)),
                pl.BlockSpec((27, cin, cout), lambda i: (0, 0, 0)),
                pl.BlockSpec((1, cout), lambda i: (0, 0))]
    res_mode = 0
    if res is not None:
        rcin = res.shape[-1]
        rf = jnp.pad(res, ((1, 1), (1, 1), (1, 1), (0, 0))).reshape(Np, rcin)
        rf = jnp.pad(rf, ((0, np_pad - Np), (0, 0)))
        if res_wb is None:
            res_mode = 1
            args += [rf]
            in_specs += [pl.BlockSpec((T, rcin), lambda i: (i, 0))]
        else:
            res_mode = 2
            rw, rb = res_wb
            args += [rf, rw.reshape(cout, rcin).T, rb.reshape(1, cout)]
            in_specs += [pl.BlockSpec((T, rcin), lambda i: (i, 0)),
                         pl.BlockSpec((rcin, cout), lambda i: (0, 0)),
                         pl.BlockSpec((1, cout), lambda i: (0, 0))]

    cfg = dict(T=T, Tw=Tw, center=center, Hp=Hp, Wp=Wp,
               cin=cin, cout=cout, res_mode=res_mode)
    out = pl.pallas_call(
        functools.partial(_conv_body, cfg=cfg),
        grid=(ntiles,),
        in_specs=in_specs,
        out_specs=pl.BlockSpec((T, cout), lambda i: (i, 0)),
        out_shape=jax.ShapeDtypeStruct((np_pad, cout), jnp.float32),
        interpret=_INTERPRET,
    )(*args)
    out = out[:Np].reshape(Dp, Hp, Wp, cout)
    if mode == 'down':
        return out[2:D + 1:2, 2:H + 1:2, 2:W + 1:2]
    return out[1:1 + D, 1:1 + H, 1:1 + W]


def _resblock(h, p):
    h1 = _run_conv(h, p['c1'], gn=p['n1'], silu=True)
    return _run_conv(h1, p['c2'], gn=p['n2'], silu=True,
                     res=h, res_wb=p.get('short'))


def _upsample2(h):
    return jnp.repeat(jnp.repeat(jnp.repeat(h, 2, 0), 2, 1), 2, 2)


def _xconv3d(x, wb, stride=1, pad=1):
    w, b = wb
    out = jax.lax.conv_general_dilated(
        x, w, (stride, stride, stride), [(pad, pad)] * 3,
        dimension_numbers=('NCDHW', 'OIDHW', 'NCDHW'))
    return out + b[None, :, None, None, None]


def _xgroup_norm(x, gb):
    g, b = gb
    B, C, D, H, W = x.shape
    xg = x.reshape(B, GROUPS, C // GROUPS, D, H, W)
    m = xg.mean(axis=(2, 3, 4, 5), keepdims=True)
    v = ((xg - m) ** 2).mean(axis=(2, 3, 4, 5), keepdims=True)
    xg = (xg - m) / jnp.sqrt(v + EPS)
    x = xg.reshape(B, C, D, H, W)
    return x * g[None, :, None, None, None] + b[None, :, None, None, None]


def _xresblock(x, p):
    h = _xgroup_norm(x, p['n1'])
    h = jax.nn.silu(h)
    h = _xconv3d(h, p['c1'])
    h = _xgroup_norm(h, p['n2'])
    h = jax.nn.silu(h)
    h = _xconv3d(h, p['c2'])
    if 'short' in p:
        x = _xconv3d(x, p['short'], pad=0)
    return x + h


def _encoder(x, p):
    h = _xconv3d(x, p['conv_in'])
    for lvl in p['levels']:
        for rp in lvl['res']:
            h = _xresblock(h, rp)
        if lvl['down'] is not None:
            hd = jnp.pad(h, ((0, 0), (0, 0), (0, 1), (0, 1), (0, 1)))
            h = _xconv3d(hd, lvl['down'], stride=2, pad=0)
    h = _xgroup_norm(h, p['norm_out'])
    h = _xconv3d(h, p['conv_out'])
    return h


def _vq_body(z_ref, wq_ref, bq_ref, embt_ref, emb_ref, wpq_ref, bpq_ref,
             q_ref, loss_ref, *more, n, K, want_idx=False):
    z4 = z_ref[...]
    z = _mm(z4, wq_ref[...]) + bq_ref[...]
    embt = embt_ref[...]
    en = jnp.sum(embt * embt, axis=0, keepdims=True)
    zn = jnp.sum(z * z, axis=1, keepdims=True)
    ze = _mm(z, embt)
    d = zn + en - 2.0 * ze
    dmin = jnp.min(d, axis=1, keepdims=True)
    iota = jax.lax.broadcasted_iota(jnp.int32, (n, K), 1).astype(jnp.float32)
    idx = jnp.min(jnp.where(d <= dmin, iota, float(K)), axis=1, keepdims=True)
    onehot = (iota == idx).astype(jnp.float32)
    zq = jax.lax.dot_general(onehot, emb_ref[...], (((1,), (0,)), ((), ())),
                             preferred_element_type=jnp.float32)
    df = zq - z
    loss_ref[...] = ((1.0 + BETA) * jnp.mean(df * df)).reshape(1, 1)
    q_ref[...] = _mm(zq, wpq_ref[...]) + bpq_ref[...]
    if want_idx:
        more[0][...] = idx


def _run_vq(z4, quant_wb, emb, post_wb):
    D, H, W, lat = z4.shape
    n = D * H * W
    K, edim = emb.shape
    wq, bq = quant_wb
    wpq, bpq = post_wb
    q, loss = pl.pallas_call(
        functools.partial(_vq_body, n=n, K=K),
        out_shape=(jax.ShapeDtypeStruct((n, lat), jnp.float32),
                   jax.ShapeDtypeStruct((1, 1), jnp.float32)),
        interpret=_INTERPRET,
    )(z4.reshape(n, lat), wq.reshape(edim, lat).T, bq.reshape(1, edim),
      emb.T, emb, wpq.reshape(lat, edim).T, bpq.reshape(1, lat))
    return q.reshape(D, H, W, lat), loss.reshape(())


def kernel(x, params):
    z = _encoder(x, params['enc'])
    h = jnp.transpose(z[0], (1, 2, 3, 0))

    q, diff = _run_vq(h, params['quant_conv'], params['embedding'],
                      params['post_quant_conv'])

    dec = params['dec']
    h = _run_conv(q, dec['conv_in'])
    for lvl in dec['levels']:
        for rp in lvl['res']:
            h = _resblock(h, rp)
        if lvl['up'] is not None:
            h = _run_conv(_upsample2(h), lvl['up'])
    h = _run_conv(h, dec['conv_out'], gn=dec['norm_out'], silu=False)

    return h.reshape(1, 1, 32, 32, 32), diff


# speedup vs baseline: 1.1166x; 1.1166x over previous
import functools

import jax
import jax.numpy as jnp
from jax.experimental import pallas as pl

GROUPS = 16
EPS = 1e-06
BETA = 0.25

_INTERPRET = False


def _mm(a, b):
    return jax.lax.dot_general(
        a.astype(jnp.bfloat16), b.astype(jnp.bfloat16),
        (((1,), (0,)), ((), ())), preferred_element_type=jnp.float32)


def _ceil_to(a, b):
    return -(-a // b) * b


def _stats_body(x_ref, o_ref):
    i = pl.program_id(0)

    @pl.when(i == 0)
    def _():
        o_ref[...] = jnp.zeros_like(o_ref)

    x = x_ref[...]
    o_ref[0:1, :] = o_ref[0:1, :] + jnp.sum(x, axis=0, keepdims=True)
    o_ref[1:2, :] = o_ref[1:2, :] + jnp.sum(x * x, axis=0, keepdims=True)


def _gn_body(x_ref, st_ref, g_ref, gb_ref, o_ref, *, gs, n_int, cin, silu):
    s1 = st_ref[0:1, :]
    s2 = st_ref[1:2, :]
    r = jax.lax.broadcasted_iota(jnp.int32, (cin, cin), 0) // gs
    c = jax.lax.broadcasted_iota(jnp.int32, (cin, cin), 1) // gs
    avg = jnp.where(r == c, 1.0 / (gs * n_int), 0.0).astype(jnp.float32)
    m = jax.lax.dot_general(s1, avg, (((1,), (0,)), ((), ())),
                            precision=jax.lax.Precision.HIGHEST,
                            preferred_element_type=jnp.float32)
    ex2 = jax.lax.dot_general(s2, avg, (((1,), (0,)), ((), ())),
                              precision=jax.lax.Precision.HIGHEST,
                              preferred_element_type=jnp.float32)
    var = ex2 - m * m
    y = (x_ref[...] - m) * jax.lax.rsqrt(var + EPS) * g_ref[...] + gb_ref[...]
    if silu:
        y = y * jax.nn.sigmoid(y)
    o_ref[...] = y


def _gn_apply(x2d, gn, silu):
    n_int, cin = x2d.shape
    g, gb = gn
    T = min(4096, _ceil_to(n_int, 8))
    ntiles = -(-n_int // T)
    npad = ntiles * T
    xp = jnp.pad(x2d, ((0, npad - n_int), (0, 0)))
    stats = pl.pallas_call(
        _stats_body,
        grid=(ntiles,),
        in_specs=[pl.BlockSpec((T, cin), lambda i: (i, 0))],
        out_specs=pl.BlockSpec((8, cin), lambda i: (0, 0)),
        out_shape=jax.ShapeDtypeStruct((8, cin), jnp.float32),
        interpret=_INTERPRET,
    )(xp)
    y = pl.pallas_call(
        functools.partial(_gn_body, gs=cin // GROUPS, n_int=n_int,
                          cin=cin, silu=silu),
        grid=(ntiles,),
        in_specs=[pl.BlockSpec((T, cin), lambda i: (i, 0)),
                  pl.BlockSpec((8, cin), lambda i: (0, 0)),
                  pl.BlockSpec((1, cin), lambda i: (0, 0)),
                  pl.BlockSpec((1, cin), lambda i: (0, 0))],
        out_specs=pl.BlockSpec((T, cin), lambda i: (i, 0)),
        out_shape=jax.ShapeDtypeStruct((npad, cin), jnp.float32),
        interpret=_INTERPRET,
    )(xp, stats, g.reshape(1, cin), gb.reshape(1, cin))
    return y[:n_int]


def _conv_body(x_ref, w_ref, b_ref, *rest, cfg):
    T, Tw, center, cin, cout = (cfg['T'], cfg['Tw'], cfg['center'],
                                cfg['cin'], cfg['cout'])
    G = cfg['G']
    rest = list(rest)
    out_ref = rest.pop()
    i = pl.program_id(0)
    xw = x_ref[pl.ds(i * T, Tw), :].astype(jnp.bfloat16)

    Hp, Wp = cfg['Hp'], cfg['Wp']
    rels = [kd * Hp * Wp + kh * Wp + kw
            for kd in (-1, 0, 1) for kh in (-1, 0, 1) for kw in (-1, 0, 1)]
    rels += [0] * (-len(rels) % G)
    acc = None
    for g0 in range(0, len(rels), G):
        xs = [jax.lax.slice(xw, (center + rels[t], 0),
                            (center + rels[t] + T, cin))
              for t in range(g0, g0 + G)]
        xg = xs[0] if G == 1 else jnp.concatenate(xs, axis=1)
        wg = w_ref[g0 // G].astype(jnp.bfloat16)
        part = jax.lax.dot_general(xg, wg, (((1,), (0,)), ((), ())),
                                   preferred_element_type=jnp.float32)
        acc = part if acc is None else acc + part
    acc = acc + b_ref[...]

    if cfg['res_mode'] == 1:
        acc = acc + rest[0][...]
    elif cfg['res_mode'] == 2:
        acc = acc + _mm(rest[0][...], rest[1][...]) + rest[2][...]

    out_ref[...] = acc


def _run_conv(x, wb, gn=None, silu=False, res=None, res_wb=None, mode='same'):
    D, H, W, cin = x.shape
    Dp, Hp, Wp = D + 2, H + 2, W + 2
    Np = Dp * Hp * Wp
    center = Hp * Wp + Wp + 1
    w, b = wb
    cout = w.shape[0]

    if gn is not None:
        x = _gn_apply(x.reshape(D * H * W, cin), gn, silu).reshape(x.shape)

    T = min(4096, _ceil_to(Np, 8))
    ntiles = -(-Np // T)
    np_pad = ntiles * T
    Tw = _ceil_to(T + 2 * center, 8)
    L = (ntiles - 1) * T + Tw

    xf = jnp.pad(x, ((1, 1), (1, 1), (1, 1), (0, 0))).reshape(Np, cin)
    xe = jnp.pad(xf, ((center, L - center - Np), (0, 0)))

    G = max(1, min(27, 128 // cin))
    ng = -(-27 // G)
    wt = jnp.transpose(w, (2, 3, 4, 1, 0)).reshape(27, cin, cout)
    wt = jnp.pad(wt, ((0, ng * G - 27), (0, 0), (0, 0)))
    wt = wt.reshape(ng, G * cin, cout)

    args = [xe, wt, b.reshape(1, cout)]
    in_specs = [pl.BlockSpec((L, cin), lambda i: (0, 0)),
                pl.BlockSpec((ng, G * cin, cout), lambda i: (0, 0, 0)),
                pl.BlockSpec((1, cout), lambda i: (0, 0))]
    res_mode = 0
    if res is not None:
        rcin = res.shape[-1]
        rf = jnp.pad(res, ((1, 1), (1, 1), (1, 1), (0, 0))).reshape(Np, rcin)
        rf = jnp.pad(rf, ((0, np_pad - Np), (0, 0)))
        if res_wb is None:
            res_mode = 1
            args += [rf]
            in_specs += [pl.BlockSpec((T, rcin), lambda i: (i, 0))]
        else:
            res_mode = 2
            rw, rb = res_wb
            args += [rf, rw.reshape(cout, rcin).T, rb.reshape(1, cout)]
            in_specs += [pl.BlockSpec((T, rcin), lambda i: (i, 0)),
                         pl.BlockSpec((rcin, cout), lambda i: (0, 0)),
                         pl.BlockSpec((1, cout), lambda i: (0, 0))]

    cfg = dict(T=T, Tw=Tw, center=center, Hp=Hp, Wp=Wp,
               cin=cin, cout=cout, res_mode=res_mode, G=G)
    out = pl.pallas_call(
        functools.partial(_conv_body, cfg=cfg),
        grid=(ntiles,),
        in_specs=in_specs,
        out_specs=pl.BlockSpec((T, cout), lambda i: (i, 0)),
        out_shape=jax.ShapeDtypeStruct((np_pad, cout), jnp.float32),
        interpret=_INTERPRET,
    )(*args)
    out = out[:Np].reshape(Dp, Hp, Wp, cout)
    if mode == 'down':
        return out[2:D + 1:2, 2:H + 1:2, 2:W + 1:2]
    return out[1:1 + D, 1:1 + H, 1:1 + W]


def _resblock(h, p):
    h1 = _run_conv(h, p['c1'], gn=p['n1'], silu=True)
    return _run_conv(h1, p['c2'], gn=p['n2'], silu=True,
                     res=h, res_wb=p.get('short'))


def _upsample2(h):
    return jnp.repeat(jnp.repeat(jnp.repeat(h, 2, 0), 2, 1), 2, 2)


def _xconv3d(x, wb, stride=1, pad=1):
    w, b = wb
    out = jax.lax.conv_general_dilated(
        x, w, (stride, stride, stride), [(pad, pad)] * 3,
        dimension_numbers=('NCDHW', 'OIDHW', 'NCDHW'))
    return out + b[None, :, None, None, None]


def _xgroup_norm(x, gb):
    g, b = gb
    B, C, D, H, W = x.shape
    xg = x.reshape(B, GROUPS, C // GROUPS, D, H, W)
    m = xg.mean(axis=(2, 3, 4, 5), keepdims=True)
    v = ((xg - m) ** 2).mean(axis=(2, 3, 4, 5), keepdims=True)
    xg = (xg - m) / jnp.sqrt(v + EPS)
    x = xg.reshape(B, C, D, H, W)
    return x * g[None, :, None, None, None] + b[None, :, None, None, None]


def _xresblock(x, p):
    h = _xgroup_norm(x, p['n1'])
    h = jax.nn.silu(h)
    h = _xconv3d(h, p['c1'])
    h = _xgroup_norm(h, p['n2'])
    h = jax.nn.silu(h)
    h = _xconv3d(h, p['c2'])
    if 'short' in p:
        x = _xconv3d(x, p['short'], pad=0)
    return x + h


def _encoder(x, p):
    h = _xconv3d(x, p['conv_in'])
    for lvl in p['levels']:
        for rp in lvl['res']:
            h = _xresblock(h, rp)
        if lvl['down'] is not None:
            hd = jnp.pad(h, ((0, 0), (0, 0), (0, 1), (0, 1), (0, 1)))
            h = _xconv3d(hd, lvl['down'], stride=2, pad=0)
    h = _xgroup_norm(h, p['norm_out'])
    h = _xconv3d(h, p['conv_out'])
    return h


def _vq_body(z_ref, wq_ref, bq_ref, embt_ref, emb_ref, wpq_ref, bpq_ref,
             q_ref, loss_ref, *more, n, K, want_idx=False):
    z4 = z_ref[...]
    z = _mm(z4, wq_ref[...]) + bq_ref[...]
    embt = embt_ref[...]
    en = jnp.sum(embt * embt, axis=0, keepdims=True)
    zn = jnp.sum(z * z, axis=1, keepdims=True)
    ze = _mm(z, embt)
    d = zn + en - 2.0 * ze
    dmin = jnp.min(d, axis=1, keepdims=True)
    iota = jax.lax.broadcasted_iota(jnp.int32, (n, K), 1).astype(jnp.float32)
    idx = jnp.min(jnp.where(d <= dmin, iota, float(K)), axis=1, keepdims=True)
    onehot = (iota == idx).astype(jnp.float32)
    zq = jax.lax.dot_general(onehot, emb_ref[...], (((1,), (0,)), ((), ())),
                             preferred_element_type=jnp.float32)
    df = zq - z
    loss_ref[...] = ((1.0 + BETA) * jnp.mean(df * df)).reshape(1, 1)
    q_ref[...] = _mm(zq, wpq_ref[...]) + bpq_ref[...]
    if want_idx:
        more[0][...] = idx


def _run_vq(z4, quant_wb, emb, post_wb):
    D, H, W, lat = z4.shape
    n = D * H * W
    K, edim = emb.shape
    wq, bq = quant_wb
    wpq, bpq = post_wb
    q, loss = pl.pallas_call(
        functools.partial(_vq_body, n=n, K=K),
        out_shape=(jax.ShapeDtypeStruct((n, lat), jnp.float32),
                   jax.ShapeDtypeStruct((1, 1), jnp.float32)),
        interpret=_INTERPRET,
    )(z4.reshape(n, lat), wq.reshape(edim, lat).T, bq.reshape(1, edim),
      emb.T, emb, wpq.reshape(lat, edim).T, bpq.reshape(1, lat))
    return q.reshape(D, H, W, lat), loss.reshape(())


def kernel(x, params):
    z = _encoder(x, params['enc'])
    h = jnp.transpose(z[0], (1, 2, 3, 0))

    q, diff = _run_vq(h, params['quant_conv'], params['embedding'],
                      params['post_quant_conv'])

    dec = params['dec']
    h = _run_conv(q, dec['conv_in'])
    for lvl in dec['levels']:
        for rp in lvl['res']:
            h = _resblock(h, rp)
        if lvl['up'] is not None:
            h = _run_conv(_upsample2(h), lvl['up'])
    h = _run_conv(h, dec['conv_out'], gn=dec['norm_out'], silu=False)

    return h.reshape(1, 1, 32, 32, 32), diff
